# Initial kernel scaffold; baseline (speedup 1.0000x reference)
import jax
import jax.numpy as jnp
from jax.experimental import pallas as pl


def kernel(z_loga):
    raise NotImplementedError("write your pallas kernel here")


# speedup vs baseline: 66.7495x; 66.7495x over previous
import functools

import jax
import jax.numpy as jnp
from jax.experimental import pallas as pl
from jax.experimental.pallas import tpu as pltpu

_TEMPERATURE = 2.0 / 3.0
_MAGIC = 0.8
_NROWS = 128
_NCOLS = 8192
_NZEROS = _NCOLS - 4096


def _mask_body(z_ref, o_ref):
    z = z_ref[...]
    y = jax.lax.bitcast_convert_type(z, jnp.int32)
    v = jnp.where(y < 0, ~y, y ^ jnp.int32(-(2**31)))
    u = jax.lax.bitcast_convert_type(v, jnp.uint32)

    k = jnp.int32(_NZEROS)

    def bit_step(i, p):
        b = jnp.uint32(31) - jnp.uint32(i)
        cand = p | (jnp.uint32(1) << b)
        cnt = jnp.sum((u < cand).astype(jnp.int32), axis=1, keepdims=True)
        return jnp.where(cnt < k, cand, p)

    p0 = jnp.zeros((_NROWS, 1), dtype=jnp.uint32)
    thresh = jax.lax.fori_loop(0, 32, bit_step, p0)

    lt = u < thresh
    eq = u == thresh
    count_less = jnp.sum(lt.astype(jnp.int32), axis=1, keepdims=True)
    need = k - count_less

    col = jax.lax.broadcasted_iota(jnp.int32, (_NROWS, _NCOLS), 1)

    def idx_step(i, m):
        b = jnp.int32(12) - jnp.int32(i)
        cand = m | (jnp.int32(1) << b)
        cnt = jnp.sum((eq & (col < cand)).astype(jnp.int32), axis=1,
                      keepdims=True)
        return jnp.where(cnt < need, cand, m)

    m0 = jnp.zeros((_NROWS, 1), dtype=jnp.int32)
    m = jax.lax.fori_loop(0, 13, idx_step, m0)

    zero = lt | (eq & (col <= m))
    sig = jax.nn.sigmoid(z * jnp.float32(_MAGIC / _TEMPERATURE))
    o_ref[...] = jnp.where(zero, jnp.float32(0.0), sig)


@jax.jit
def kernel(z_loga):
    return pl.pallas_call(
        _mask_body,
        out_shape=jax.ShapeDtypeStruct((_NROWS, _NCOLS), jnp.float32),
    )(z_loga)
